# baseline (speedup 1.0000x reference)
import dataclasses
import functools

import jax
import jax.numpy as jnp
from jax import lax
from jax.experimental import pallas as pl
from jax.experimental.pallas import tpu as pltpu
from jax.experimental.pallas import tpu_sc as plsc

N0, N1, N2 = 10000, 160000, 50000
D, F, OUT, B = 256, 20, 64, 64
FP = 32
NW = 32
CH = 128
NPAD = {N0: 10240, N1: 161792, N2: 50176}
ROWB = 512


@functools.lru_cache(maxsize=None)
def _make_spmm(nnzp, rb, nblk):
    mesh = plsc.VectorSubcoreMesh(core_axis_name="c", subcore_axis_name="s")
    cp = pltpu.CompilerParams(use_tc_tiling_on_sc=False)

    @functools.partial(
        pl.kernel,
        out_type=jax.ShapeDtypeStruct((nblk * rb, FP), jnp.float32),
        mesh=mesh,
        compiler_params=cp,
        scratch_types=[
            pltpu.VMEM((rb, FP), jnp.float32),
            pltpu.VMEM((rb, FP), jnp.float32),
            pltpu.VMEM((CH, FP), jnp.float32),
            pltpu.VMEM((CH, FP), jnp.float32),
            pltpu.VMEM((CH,), jnp.int32),
            pltpu.VMEM((CH,), jnp.int32),
            pltpu.VMEM((CH + 16,), jnp.int32),
            pltpu.VMEM((CH + 16,), jnp.int32),
            pltpu.VMEM((CH + 16,), jnp.float32),
            pltpu.VMEM((CH + 16,), jnp.float32),
            pltpu.VMEM((nblk + 16,), jnp.int32),
            pltpu.VMEM((nblk + 16,), jnp.int32),
            pltpu.SemaphoreType.DMA,
            pltpu.SemaphoreType.DMA,
            pltpu.SemaphoreType.DMA,
            pltpu.SemaphoreType.DMA,
            pltpu.SemaphoreType.DMA,
            pltpu.SemaphoreType.DMA,
        ],
    )
    def k(rows_h, cols_h, vals_h, ck0_h, ck1_h, xw_h, out_h,
          acc, acc2, gbufA, gbufB, colA, colB, rowA, rowB, valA, valB,
          ck0_s, ck1_s, semiA, semiB, semcA, semcB, semgA, semgB):
        wid = lax.axis_index("s") * 2 + lax.axis_index("c")
        pltpu.async_copy(ck0_h, ck0_s.at[pl.ds(0, nblk)], semiA).wait()
        pltpu.async_copy(ck1_h, ck1_s.at[pl.ds(0, nblk)], semiA).wait()
        zz = jnp.zeros((16,), jnp.float32)
        bufs = ((gbufA, colA, rowA, valA, semiA, semcA, semgA),
                (gbufB, colB, rowB, valB, semiB, semcB, semgB))

        def issue(kk, bs):
            gbuf, colv, rowv, valv, semi, semc, semg = bs
            o = kk * CH
            pltpu.async_copy(cols_h.at[pl.ds(o, CH)], colv, semc).wait()
            pltpu.async_copy(rows_h.at[pl.ds(o, CH)],
                             rowv.at[pl.ds(0, CH)], semi)
            pltpu.async_copy(vals_h.at[pl.ds(o, CH)],
                             valv.at[pl.ds(0, CH)], semi)
            pltpu.make_async_copy(xw_h.at[colv], gbuf, semg).start()

        def compute(base, bs):
            gbuf, colv, rowv, valv, semi, semc, semg = bs
            pltpu.make_async_copy(rows_h.at[pl.ds(0, CH)],
                                  rowv.at[pl.ds(0, CH)], semi).wait()
            pltpu.make_async_copy(vals_h.at[pl.ds(0, CH)],
                                  valv.at[pl.ds(0, CH)], semi).wait()
            pltpu.make_async_copy(xw_h.at[colv], gbuf, semg).wait()

            @pl.loop(0, CH, step=16)
            def _(q):
                rs16 = rowv[pl.ds(q, 16)]
                ok16 = (rs16 >= base) & (rs16 < base + rb)
                v16 = jnp.where(ok16, valv[pl.ds(q, 16)], 0.0)
                li16 = jnp.minimum(jnp.maximum(rs16 - base, 0), rb - 1)
                for i in range(16):
                    li = li16[i]
                    vv = jnp.full((16,), v16[i], jnp.float32)
                    a = acc if i % 2 == 0 else acc2
                    a[li, pl.ds(0, 16)] += gbuf[q + i, pl.ds(0, 16)] * vv
                    a[li, pl.ds(16, 16)] += gbuf[q + i, pl.ds(16, 16)] * vv

        for t in range(nblk // NW):
            b = wid + t * NW
            base = b * rb

            @pl.loop(0, rb, step=8)
            def _(r):
                for i in range(8):
                    acc[r + i, pl.ds(0, 16)] = zz
                    acc[r + i, pl.ds(16, 16)] = zz
                    acc2[r + i, pl.ds(0, 16)] = zz
                    acc2[r + i, pl.ds(16, 16)] = zz

            c0 = ck0_s[pl.ds(b, 16)][0]
            c1 = ck1_s[pl.ds(b, 16)][0]

            @pl.when(c0 < c1)
            def _():
                issue(c0, bufs[0])

            @pl.loop(c0, c1, step=2)
            def _(kk0):
                @pl.when(kk0 + 1 < c1)
                def _():
                    issue(kk0 + 1, bufs[1])
                compute(base, bufs[0])

                @pl.when(kk0 + 1 < c1)
                def _():
                    @pl.when(kk0 + 2 < c1)
                    def _():
                        issue(kk0 + 2, bufs[0])
                    compute(base, bufs[1])

            @pl.loop(0, rb, step=2)
            def _(r):
                for i in range(2):
                    acc[r + i, pl.ds(0, 16)] += acc2[r + i, pl.ds(0, 16)]
                    acc[r + i, pl.ds(16, 16)] += acc2[r + i, pl.ds(16, 16)]

            pltpu.sync_copy(acc, out_h.at[pl.ds(base, rb)])

    return k


def _spmm(rows, cols, vals, xw, n_dst):
    nnz = rows.shape[0]
    if n_dst == N1:
        nblk, rb = (256, 632) if nnz > 600000 else (128, 1264)
    elif n_dst == N0:
        nblk, rb = 64, 160
    else:
        nblk, rb = 64, 784
    nnzp = ((nnz + CH - 1) // CH) * CH
    if nnzp != nnz:
        pad = nnzp - nnz
        rows = jnp.concatenate([rows, jnp.full((pad,), 1 << 28, jnp.int32)])
        cols = jnp.concatenate([cols, jnp.zeros((pad,), jnp.int32)])
        vals = jnp.concatenate([vals, jnp.zeros((pad,), jnp.float32)])
    off = jnp.searchsorted(rows, jnp.arange(nblk + 1, dtype=jnp.int32) * rb,
                           side="left").astype(jnp.int32)
    ck0 = off[:nblk] // CH
    ck1 = (off[1:] + CH - 1) // CH
    return _make_spmm(nnzp, rb, nblk)(rows, cols, vals, ck0, ck1, xw)


def _transform1(x, wcat, bcat, nops):
    n, kdim = x.shape

    def body(x_ref, w_ref, b_ref, *outs):
        y = jnp.dot(x_ref[...], w_ref[...], preferred_element_type=jnp.float32)
        y = y + b_ref[...]
        for i, o_ref in enumerate(outs):
            o_ref[...] = y[:, i * FP:(i + 1) * FP]

    grid = (pl.cdiv(n, ROWB),)
    return pl.pallas_call(
        body,
        grid=grid,
        in_specs=[
            pl.BlockSpec((ROWB, kdim), lambda i: (i, 0)),
            pl.BlockSpec((kdim, 4 * FP), lambda i: (0, 0)),
            pl.BlockSpec((1, 4 * FP), lambda i: (0, 0)),
        ],
        out_specs=[pl.BlockSpec((ROWB, FP), lambda i: (i, 0))] * nops,
        out_shape=[jax.ShapeDtypeStruct((n, FP), jnp.float32)] * nops,
    )(x, wcat, bcat)


def _combine_transform(parts, scale, wcat, bcat, nops):
    n = parts[0].shape[0]

    def body(*refs):
        ins = refs[:len(parts)]
        s = ins[0][...]
        for r in ins[1:len(parts)]:
            s = s + r[...]
        xc = scale * jnp.maximum(s, 0.0)
        if nops:
            w_ref = refs[len(parts)]
            b_ref = refs[len(parts) + 1]
            outs = refs[len(parts) + 2:]
            outs[0][...] = xc
            y = jnp.dot(xc, w_ref[...], preferred_element_type=jnp.float32)
            y = y + b_ref[...]
            for i in range(nops):
                outs[1 + i][...] = y[:, i * FP:(i + 1) * FP]
        else:
            refs[len(parts)][...] = xc

    grid = (n // ROWB,)
    in_specs = [pl.BlockSpec((ROWB, FP), lambda i: (i, 0))] * len(parts)
    if nops:
        in_specs += [
            pl.BlockSpec((FP, 4 * FP), lambda i: (0, 0)),
            pl.BlockSpec((1, 4 * FP), lambda i: (0, 0)),
        ]
        args = list(parts) + [wcat, bcat]
    else:
        args = list(parts)
    n_out = 1 + nops
    res = pl.pallas_call(
        body, grid=grid, in_specs=in_specs,
        out_specs=[pl.BlockSpec((ROWB, FP), lambda i: (i, 0))] * n_out,
        out_shape=[jax.ShapeDtypeStruct((n, FP), jnp.float32)] * n_out,
    )(*args)
    return (res[0], list(res[1:])) if nops else (res[0], [])


def _pool(xs, seg3d):
    n = xs[0].shape[0]

    def body(a_ref, b_ref, c_ref, s_ref, sums_ref, cnt_ref):
        seg = s_ref[0, 0, :]
        oh = (seg[:, None] == lax.broadcasted_iota(jnp.int32, (ROWB, B), 1))
        oh = oh.astype(jnp.float32)
        x = jnp.concatenate([a_ref[...], b_ref[...], c_ref[...]], axis=1)
        ps = lax.dot_general(oh, x, (((0,), (0,)), ((), ())),
                             preferred_element_type=jnp.float32)
        pc = lax.dot_general(oh, jnp.ones((ROWB, 8), jnp.float32),
                             (((0,), (0,)), ((), ())),
                             preferred_element_type=jnp.float32)

        @pl.when(pl.program_id(0) == 0)
        def _():
            sums_ref[...] = jnp.zeros_like(sums_ref)
            cnt_ref[...] = jnp.zeros_like(cnt_ref)

        sums_ref[...] += ps
        cnt_ref[...] += pc

    grid = (n // ROWB,)
    return pl.pallas_call(
        body,
        grid=grid,
        in_specs=[pl.BlockSpec((ROWB, FP), lambda i: (i, 0))] * 3
        + [pl.BlockSpec((1, 1, ROWB), lambda i: (i, 0, 0))],
        out_specs=[pl.BlockSpec((B, 3 * FP), lambda i: (0, 0)),
                   pl.BlockSpec((B, 8), lambda i: (0, 0))],
        out_shape=[jax.ShapeDtypeStruct((B, 3 * FP), jnp.float32),
                   jax.ShapeDtypeStruct((B, 8), jnp.float32)],
    )(*xs, seg3d)


def _head(sums, cnts, wc, bc, wo_parts, bo):
    def body(s0, s1, s2, c0, c1, c2, wc_ref, bc_ref, w0, w1, w2, bo_ref, o_ref):
        logits = bo_ref[...]
        for s_ref, c_ref, w_ref in ((s0, c0, w0), (s1, c1, w1), (s2, c2, w2)):
            cnt = jnp.maximum(c_ref[...][:, 0:1], 1.0)
            g = s_ref[...] / cnt
            p = jnp.dot(g, wc_ref[...], preferred_element_type=jnp.float32)
            p = p + bc_ref[...]
            logits = logits + jnp.dot(p, w_ref[...],
                                      preferred_element_type=jnp.float32)
        m = jnp.max(logits, axis=1, keepdims=True)
        e = jnp.exp(logits - m)
        o_ref[...] = e / jnp.sum(e, axis=1, keepdims=True)

    return pl.pallas_call(
        body,
        out_shape=jax.ShapeDtypeStruct((B, OUT), jnp.float32),
    )(sums[0], sums[1], sums[2], cnts[0], cnts[1], cnts[2], wc, bc,
      wo_parts[0], wo_parts[1], wo_parts[2], bo)


def _pack_w(ws, kdim):
    w = jnp.zeros((kdim, 4 * FP), jnp.float32)
    bvec = jnp.zeros((1, 4 * FP), jnp.float32)
    for i, (wi, bi) in enumerate(ws):
        kin = wi.shape[0]
        w = w.at[:kin, i * FP:i * FP + F].set(wi)
        bvec = bvec.at[0, i * FP:i * FP + F].set(bi)
    return w, bvec


def _pad_rows(x, npad):
    n = x.shape[0]
    if n == npad:
        return x
    return jnp.concatenate(
        [x, jnp.zeros((npad - n,) + x.shape[1:], x.dtype)], axis=0)


def kernel(X0, X1, X2, L0_rows, L0_cols, L0_vals, L1_rows, L1_cols, L1_vals,
           L2_rows, L2_cols, L2_vals, B2D3_rows, B2D3_cols, B2D3_vals,
           D2B1TD1inv_rows, D2B1TD1inv_cols, D2B1TD1inv_vals,
           D1invB1_rows, D1invB1_cols, D1invB1_vals,
           B2TD2inv_rows, B2TD2inv_cols, B2TD2inv_vals,
           batch0, batch1, batch2, params):
    p = params
    sp = {
        "L0": (L0_rows, L0_cols, L0_vals, N0),
        "L1": (L1_rows, L1_cols, L1_vals, N1),
        "L2": (L2_rows, L2_cols, L2_vals, N2),
        "B2D3": (B2D3_rows, B2D3_cols, B2D3_vals, N1),
        "D2B1TD1inv": (D2B1TD1inv_rows, D2B1TD1inv_cols, D2B1TD1inv_vals, N1),
        "D1invB1": (D1invB1_rows, D1invB1_cols, D1invB1_vals, N0),
        "B2TD2inv": (B2TD2inv_rows, B2TD2inv_cols, B2TD2inv_vals, N2),
    }

    def run_spmms(t0, t1, t2):
        n2n = _spmm(*sp["L0"][:3], t0[0], sp["L0"][3])
        n2e = _spmm(*sp["D2B1TD1inv"][:3], t0[1], sp["D2B1TD1inv"][3])
        e2n = _spmm(*sp["D1invB1"][:3], t1[0], sp["D1invB1"][3])
        e2e = _spmm(*sp["L1"][:3], t1[1], sp["L1"][3])
        e2t = _spmm(*sp["B2TD2inv"][:3], t1[2], sp["B2TD2inv"][3])
        t2t = _spmm(*sp["L2"][:3], t2[0], sp["L2"][3])
        t2e = _spmm(*sp["B2D3"][:3], t2[1], sp["B2D3"][3])
        return (n2n, e2n), (e2e, n2e, t2e), (t2t, e2t)

    def packed(layer, keys, kdim):
        return _pack_w([p[layer][k] for k in keys], kdim)

    w0, b0 = packed("l1", ("n2n", "n2e"), D)
    w1, b1 = packed("l1", ("e2n", "e2e", "e2t"), D)
    w2, b2 = packed("l1", ("t2t", "t2e"), D)
    t0 = _transform1(X0, w0, b0, 2)
    t1 = _transform1(X1, w1, b1, 3)
    t2 = _transform1(X2, w2, b2, 2)
    g0, g1, g2 = run_spmms(t0, t1, t2)

    xcs = []
    for layer in ("l2", "l3"):
        w0, b0 = packed(layer, ("n2n", "n2e"), FP)
        w1, b1 = packed(layer, ("e2n", "e2e", "e2t"), FP)
        w2, b2 = packed(layer, ("t2t", "t2e"), FP)
        xc0, t0 = _combine_transform(g0, 0.5, w0, b0, 2)
        xc1, t1 = _combine_transform(g1, 1.0 / 3.0, w1, b1, 3)
        xc2, t2 = _combine_transform(g2, 0.5, w2, b2, 2)
        xcs.append((xc0, xc1, xc2))
        g0, g1, g2 = run_spmms(t0, t1, t2)
    xc0_3, _ = _combine_transform(g0, 0.5, None, None, 0)
    xc1_3, _ = _combine_transform(g1, 1.0 / 3.0, None, None, 0)
    xc2_3, _ = _combine_transform(g2, 0.5, None, None, 0)
    xcs.append((xc0_3, xc1_3, xc2_3))

    def seg3d(batch, n, npad):
        s = jnp.concatenate([batch.astype(jnp.int32),
                             jnp.full((npad - n,), B, jnp.int32)])
        return s.reshape(npad // ROWB, 1, ROWB)

    sums, cnts = [], []
    for lvl, (batch, n) in enumerate(((batch0, N0), (batch1, N1), (batch2, N2))):
        npad = NPAD[n]
        xs = [xcs[0][lvl], xcs[1][lvl], xcs[2][lvl]]
        s, c = _pool(xs, seg3d(batch, n, npad))
        sums.append(s)
        cnts.append(c)

    wc_raw, bc_raw = p["combined1"]
    wc = jnp.zeros((3 * FP, OUT), jnp.float32)
    for i in range(3):
        wc = wc.at[i * FP:i * FP + F, :].set(wc_raw[i * F:(i + 1) * F, :])
    bc = bc_raw.reshape(1, OUT)
    wo_raw, bo_raw = p["output"]
    wo_parts = [wo_raw[i * OUT:(i + 1) * OUT, :] for i in range(3)]
    bo = bo_raw.reshape(1, OUT)
    return _head(sums, cnts, wc, bc, wo_parts, bo)


# speedup vs baseline: 1.0280x; 1.0280x over previous
import dataclasses
import functools

import jax
import jax.numpy as jnp
from jax import lax
from jax.experimental import pallas as pl
from jax.experimental.pallas import tpu as pltpu
from jax.experimental.pallas import tpu_sc as plsc

N0, N1, N2 = 10000, 160000, 50000
D, F, OUT, B = 256, 20, 64, 64
FP = 32
NW = 32
CH = 128
NPAD = {N0: 10240, N1: 161792, N2: 50176}
ROWB = 512


@functools.lru_cache(maxsize=None)
def _make_spmm(nnzp, rb, nblk):
    mesh = plsc.VectorSubcoreMesh(core_axis_name="c", subcore_axis_name="s")
    cp = pltpu.CompilerParams(use_tc_tiling_on_sc=False)

    @functools.partial(
        pl.kernel,
        out_type=jax.ShapeDtypeStruct((nblk * rb, FP), jnp.float32),
        mesh=mesh,
        compiler_params=cp,
        scratch_types=[
            pltpu.VMEM((rb, FP), jnp.float32),
            pltpu.VMEM((CH, FP), jnp.float32),
            pltpu.VMEM((CH, FP), jnp.float32),
            pltpu.VMEM((CH,), jnp.int32),
            pltpu.VMEM((CH,), jnp.int32),
            pltpu.VMEM((CH + 16,), jnp.int32),
            pltpu.VMEM((CH + 16,), jnp.int32),
            pltpu.VMEM((CH + 16,), jnp.float32),
            pltpu.VMEM((CH + 16,), jnp.float32),
            pltpu.VMEM((nblk + 16,), jnp.int32),
            pltpu.VMEM((nblk + 16,), jnp.int32),
            pltpu.SemaphoreType.DMA,
            pltpu.SemaphoreType.DMA,
            pltpu.SemaphoreType.DMA,
            pltpu.SemaphoreType.DMA,
            pltpu.SemaphoreType.DMA,
            pltpu.SemaphoreType.DMA,
        ],
    )
    def k(rows_h, cols_h, vals_h, ck0_h, ck1_h, xw_h, out_h,
          acc, gbufA, gbufB, colA, colB, rowA, rowB, valA, valB,
          ck0_s, ck1_s, semiA, semiB, semcA, semcB, semgA, semgB):
        wid = lax.axis_index("s") * 2 + lax.axis_index("c")
        pltpu.async_copy(ck0_h, ck0_s.at[pl.ds(0, nblk)], semiA).wait()
        pltpu.async_copy(ck1_h, ck1_s.at[pl.ds(0, nblk)], semiA).wait()
        zz = jnp.zeros((16,), jnp.float32)
        bufs = ((gbufA, colA, rowA, valA, semiA, semcA, semgA),
                (gbufB, colB, rowB, valB, semiB, semcB, semgB))

        def issue(kk, bs):
            gbuf, colv, rowv, valv, semi, semc, semg = bs
            o = kk * CH
            pltpu.async_copy(cols_h.at[pl.ds(o, CH)], colv, semc).wait()
            pltpu.async_copy(rows_h.at[pl.ds(o, CH)],
                             rowv.at[pl.ds(0, CH)], semi)
            pltpu.async_copy(vals_h.at[pl.ds(o, CH)],
                             valv.at[pl.ds(0, CH)], semi)
            pltpu.make_async_copy(xw_h.at[colv], gbuf, semg).start()

        def compute(base, bs):
            gbuf, colv, rowv, valv, semi, semc, semg = bs
            pltpu.make_async_copy(rows_h.at[pl.ds(0, CH)],
                                  rowv.at[pl.ds(0, CH)], semi).wait()
            pltpu.make_async_copy(vals_h.at[pl.ds(0, CH)],
                                  valv.at[pl.ds(0, CH)], semi).wait()
            pltpu.make_async_copy(xw_h.at[colv], gbuf, semg).wait()

            @pl.loop(0, CH, step=16)
            def _(q):
                rs16 = rowv[pl.ds(q, 16)]
                ok16 = (rs16 >= base) & (rs16 < base + rb)
                v16 = jnp.where(ok16, valv[pl.ds(q, 16)], 0.0)
                li16 = jnp.minimum(jnp.maximum(rs16 - base, 0), rb - 1)
                for i in range(16):
                    li = li16[i]
                    vv = jnp.full((16,), v16[i], jnp.float32)
                    acc[li, pl.ds(0, 16)] += gbuf[q + i, pl.ds(0, 16)] * vv
                    acc[li, pl.ds(16, 16)] += gbuf[q + i, pl.ds(16, 16)] * vv

        for t in range(nblk // NW):
            b = wid + t * NW
            base = b * rb

            @pl.loop(0, rb, step=8)
            def _(r):
                for i in range(8):
                    acc[r + i, pl.ds(0, 16)] = zz
                    acc[r + i, pl.ds(16, 16)] = zz

            c0 = ck0_s[pl.ds(b, 16)][0]
            c1 = ck1_s[pl.ds(b, 16)][0]

            @pl.when(c0 < c1)
            def _():
                issue(c0, bufs[0])

            @pl.loop(c0, c1, step=2)
            def _(kk0):
                @pl.when(kk0 + 1 < c1)
                def _():
                    issue(kk0 + 1, bufs[1])
                compute(base, bufs[0])

                @pl.when(kk0 + 1 < c1)
                def _():
                    @pl.when(kk0 + 2 < c1)
                    def _():
                        issue(kk0 + 2, bufs[0])
                    compute(base, bufs[1])

            pltpu.sync_copy(acc, out_h.at[pl.ds(base, rb)])

    return k


def _spmm(rows, cols, vals, xw, n_dst):
    nnz = rows.shape[0]
    if n_dst == N1:
        nblk, rb = (256, 632) if nnz > 600000 else (128, 1264)
    elif n_dst == N0:
        nblk, rb = 64, 160
    else:
        nblk, rb = 64, 784
    nnzp = ((nnz + CH - 1) // CH) * CH
    if nnzp != nnz:
        pad = nnzp - nnz
        rows = jnp.concatenate([rows, jnp.full((pad,), 1 << 28, jnp.int32)])
        cols = jnp.concatenate([cols, jnp.zeros((pad,), jnp.int32)])
        vals = jnp.concatenate([vals, jnp.zeros((pad,), jnp.float32)])
    off = jnp.searchsorted(rows, jnp.arange(nblk + 1, dtype=jnp.int32) * rb,
                           side="left").astype(jnp.int32)
    ck0 = off[:nblk] // CH
    ck1 = (off[1:] + CH - 1) // CH
    return _make_spmm(nnzp, rb, nblk)(rows, cols, vals, ck0, ck1, xw)


def _transform1(x, wcat, bcat, nops):
    n, kdim = x.shape

    def body(x_ref, w_ref, b_ref, *outs):
        y = jnp.dot(x_ref[...], w_ref[...], preferred_element_type=jnp.float32)
        y = y + b_ref[...]
        for i, o_ref in enumerate(outs):
            o_ref[...] = y[:, i * FP:(i + 1) * FP]

    grid = (pl.cdiv(n, ROWB),)
    return pl.pallas_call(
        body,
        grid=grid,
        in_specs=[
            pl.BlockSpec((ROWB, kdim), lambda i: (i, 0)),
            pl.BlockSpec((kdim, 4 * FP), lambda i: (0, 0)),
            pl.BlockSpec((1, 4 * FP), lambda i: (0, 0)),
        ],
        out_specs=[pl.BlockSpec((ROWB, FP), lambda i: (i, 0))] * nops,
        out_shape=[jax.ShapeDtypeStruct((n, FP), jnp.float32)] * nops,
    )(x, wcat, bcat)


def _combine_transform(parts, scale, wcat, bcat, nops):
    n = parts[0].shape[0]

    def body(*refs):
        ins = refs[:len(parts)]
        s = ins[0][...]
        for r in ins[1:len(parts)]:
            s = s + r[...]
        xc = scale * jnp.maximum(s, 0.0)
        if nops:
            w_ref = refs[len(parts)]
            b_ref = refs[len(parts) + 1]
            outs = refs[len(parts) + 2:]
            outs[0][...] = xc
            y = jnp.dot(xc, w_ref[...], preferred_element_type=jnp.float32)
            y = y + b_ref[...]
            for i in range(nops):
                outs[1 + i][...] = y[:, i * FP:(i + 1) * FP]
        else:
            refs[len(parts)][...] = xc

    grid = (n // ROWB,)
    in_specs = [pl.BlockSpec((ROWB, FP), lambda i: (i, 0))] * len(parts)
    if nops:
        in_specs += [
            pl.BlockSpec((FP, 4 * FP), lambda i: (0, 0)),
            pl.BlockSpec((1, 4 * FP), lambda i: (0, 0)),
        ]
        args = list(parts) + [wcat, bcat]
    else:
        args = list(parts)
    n_out = 1 + nops
    res = pl.pallas_call(
        body, grid=grid, in_specs=in_specs,
        out_specs=[pl.BlockSpec((ROWB, FP), lambda i: (i, 0))] * n_out,
        out_shape=[jax.ShapeDtypeStruct((n, FP), jnp.float32)] * n_out,
    )(*args)
    return (res[0], list(res[1:])) if nops else (res[0], [])


def _pool(xs, seg3d):
    n = xs[0].shape[0]

    def body(a_ref, b_ref, c_ref, s_ref, sums_ref, cnt_ref):
        seg = s_ref[0, 0, :]
        oh = (seg[:, None] == lax.broadcasted_iota(jnp.int32, (ROWB, B), 1))
        oh = oh.astype(jnp.float32)
        x = jnp.concatenate([a_ref[...], b_ref[...], c_ref[...]], axis=1)
        ps = lax.dot_general(oh, x, (((0,), (0,)), ((), ())),
                             preferred_element_type=jnp.float32)
        pc = lax.dot_general(oh, jnp.ones((ROWB, 8), jnp.float32),
                             (((0,), (0,)), ((), ())),
                             preferred_element_type=jnp.float32)

        @pl.when(pl.program_id(0) == 0)
        def _():
            sums_ref[...] = jnp.zeros_like(sums_ref)
            cnt_ref[...] = jnp.zeros_like(cnt_ref)

        sums_ref[...] += ps
        cnt_ref[...] += pc

    grid = (n // ROWB,)
    return pl.pallas_call(
        body,
        grid=grid,
        in_specs=[pl.BlockSpec((ROWB, FP), lambda i: (i, 0))] * 3
        + [pl.BlockSpec((1, 1, ROWB), lambda i: (i, 0, 0))],
        out_specs=[pl.BlockSpec((B, 3 * FP), lambda i: (0, 0)),
                   pl.BlockSpec((B, 8), lambda i: (0, 0))],
        out_shape=[jax.ShapeDtypeStruct((B, 3 * FP), jnp.float32),
                   jax.ShapeDtypeStruct((B, 8), jnp.float32)],
    )(*xs, seg3d)


def _head(sums, cnts, wc, bc, wo_parts, bo):
    def body(s0, s1, s2, c0, c1, c2, wc_ref, bc_ref, w0, w1, w2, bo_ref, o_ref):
        logits = bo_ref[...]
        for s_ref, c_ref, w_ref in ((s0, c0, w0), (s1, c1, w1), (s2, c2, w2)):
            cnt = jnp.maximum(c_ref[...][:, 0:1], 1.0)
            g = s_ref[...] / cnt
            p = jnp.dot(g, wc_ref[...], preferred_element_type=jnp.float32)
            p = p + bc_ref[...]
            logits = logits + jnp.dot(p, w_ref[...],
                                      preferred_element_type=jnp.float32)
        m = jnp.max(logits, axis=1, keepdims=True)
        e = jnp.exp(logits - m)
        o_ref[...] = e / jnp.sum(e, axis=1, keepdims=True)

    return pl.pallas_call(
        body,
        out_shape=jax.ShapeDtypeStruct((B, OUT), jnp.float32),
    )(sums[0], sums[1], sums[2], cnts[0], cnts[1], cnts[2], wc, bc,
      wo_parts[0], wo_parts[1], wo_parts[2], bo)


def _pack_w(ws, kdim):
    w = jnp.zeros((kdim, 4 * FP), jnp.float32)
    bvec = jnp.zeros((1, 4 * FP), jnp.float32)
    for i, (wi, bi) in enumerate(ws):
        kin = wi.shape[0]
        w = w.at[:kin, i * FP:i * FP + F].set(wi)
        bvec = bvec.at[0, i * FP:i * FP + F].set(bi)
    return w, bvec


def _pad_rows(x, npad):
    n = x.shape[0]
    if n == npad:
        return x
    return jnp.concatenate(
        [x, jnp.zeros((npad - n,) + x.shape[1:], x.dtype)], axis=0)


def kernel(X0, X1, X2, L0_rows, L0_cols, L0_vals, L1_rows, L1_cols, L1_vals,
           L2_rows, L2_cols, L2_vals, B2D3_rows, B2D3_cols, B2D3_vals,
           D2B1TD1inv_rows, D2B1TD1inv_cols, D2B1TD1inv_vals,
           D1invB1_rows, D1invB1_cols, D1invB1_vals,
           B2TD2inv_rows, B2TD2inv_cols, B2TD2inv_vals,
           batch0, batch1, batch2, params):
    p = params
    sp = {
        "L0": (L0_rows, L0_cols, L0_vals, N0),
        "L1": (L1_rows, L1_cols, L1_vals, N1),
        "L2": (L2_rows, L2_cols, L2_vals, N2),
        "B2D3": (B2D3_rows, B2D3_cols, B2D3_vals, N1),
        "D2B1TD1inv": (D2B1TD1inv_rows, D2B1TD1inv_cols, D2B1TD1inv_vals, N1),
        "D1invB1": (D1invB1_rows, D1invB1_cols, D1invB1_vals, N0),
        "B2TD2inv": (B2TD2inv_rows, B2TD2inv_cols, B2TD2inv_vals, N2),
    }

    def run_spmms(t0, t1, t2):
        n2n = _spmm(*sp["L0"][:3], t0[0], sp["L0"][3])
        n2e = _spmm(*sp["D2B1TD1inv"][:3], t0[1], sp["D2B1TD1inv"][3])
        e2n = _spmm(*sp["D1invB1"][:3], t1[0], sp["D1invB1"][3])
        e2e = _spmm(*sp["L1"][:3], t1[1], sp["L1"][3])
        e2t = _spmm(*sp["B2TD2inv"][:3], t1[2], sp["B2TD2inv"][3])
        t2t = _spmm(*sp["L2"][:3], t2[0], sp["L2"][3])
        t2e = _spmm(*sp["B2D3"][:3], t2[1], sp["B2D3"][3])
        return (n2n, e2n), (e2e, n2e, t2e), (t2t, e2t)

    def packed(layer, keys, kdim):
        return _pack_w([p[layer][k] for k in keys], kdim)

    w0, b0 = packed("l1", ("n2n", "n2e"), D)
    w1, b1 = packed("l1", ("e2n", "e2e", "e2t"), D)
    w2, b2 = packed("l1", ("t2t", "t2e"), D)
    t0 = _transform1(X0, w0, b0, 2)
    t1 = _transform1(X1, w1, b1, 3)
    t2 = _transform1(X2, w2, b2, 2)
    g0, g1, g2 = run_spmms(t0, t1, t2)

    xcs = []
    for layer in ("l2", "l3"):
        w0, b0 = packed(layer, ("n2n", "n2e"), FP)
        w1, b1 = packed(layer, ("e2n", "e2e", "e2t"), FP)
        w2, b2 = packed(layer, ("t2t", "t2e"), FP)
        xc0, t0 = _combine_transform(g0, 0.5, w0, b0, 2)
        xc1, t1 = _combine_transform(g1, 1.0 / 3.0, w1, b1, 3)
        xc2, t2 = _combine_transform(g2, 0.5, w2, b2, 2)
        xcs.append((xc0, xc1, xc2))
        g0, g1, g2 = run_spmms(t0, t1, t2)
    xc0_3, _ = _combine_transform(g0, 0.5, None, None, 0)
    xc1_3, _ = _combine_transform(g1, 1.0 / 3.0, None, None, 0)
    xc2_3, _ = _combine_transform(g2, 0.5, None, None, 0)
    xcs.append((xc0_3, xc1_3, xc2_3))

    def seg3d(batch, n, npad):
        s = jnp.concatenate([batch.astype(jnp.int32),
                             jnp.full((npad - n,), B, jnp.int32)])
        return s.reshape(npad // ROWB, 1, ROWB)

    sums, cnts = [], []
    for lvl, (batch, n) in enumerate(((batch0, N0), (batch1, N1), (batch2, N2))):
        npad = NPAD[n]
        xs = [xcs[0][lvl], xcs[1][lvl], xcs[2][lvl]]
        s, c = _pool(xs, seg3d(batch, n, npad))
        sums.append(s)
        cnts.append(c)

    wc_raw, bc_raw = p["combined1"]
    wc = jnp.zeros((3 * FP, OUT), jnp.float32)
    for i in range(3):
        wc = wc.at[i * FP:i * FP + F, :].set(wc_raw[i * F:(i + 1) * F, :])
    bc = bc_raw.reshape(1, OUT)
    wo_raw, bo_raw = p["output"]
    wo_parts = [wo_raw[i * OUT:(i + 1) * OUT, :] for i in range(3)]
    bo = bo_raw.reshape(1, OUT)
    return _head(sums, cnts, wc, bc, wo_parts, bo)


# speedup vs baseline: 1.0773x; 1.0480x over previous
import dataclasses
import functools

import jax
import jax.numpy as jnp
from jax import lax
from jax.experimental import pallas as pl
from jax.experimental.pallas import tpu as pltpu
from jax.experimental.pallas import tpu_sc as plsc

N0, N1, N2 = 10000, 160000, 50000
D, F, OUT, B = 256, 20, 64, 64
FP = 32
NW = 32
CH = 256
NPAD = {N0: 10240, N1: 161792, N2: 50176}
ROWB = 512


@functools.lru_cache(maxsize=None)
def _make_spmm(nnzp, rb, nblk):
    mesh = plsc.VectorSubcoreMesh(core_axis_name="c", subcore_axis_name="s")
    cp = pltpu.CompilerParams(use_tc_tiling_on_sc=False)

    @functools.partial(
        pl.kernel,
        out_type=jax.ShapeDtypeStruct((nblk * rb, FP), jnp.float32),
        mesh=mesh,
        compiler_params=cp,
        scratch_types=[
            pltpu.VMEM((rb, FP), jnp.float32),
            pltpu.VMEM((CH, FP), jnp.float32),
            pltpu.VMEM((CH, FP), jnp.float32),
            pltpu.VMEM((CH,), jnp.int32),
            pltpu.VMEM((CH,), jnp.int32),
            pltpu.VMEM((CH + 16,), jnp.int32),
            pltpu.VMEM((CH + 16,), jnp.int32),
            pltpu.VMEM((CH + 16,), jnp.float32),
            pltpu.VMEM((CH + 16,), jnp.float32),
            pltpu.VMEM((nblk + 16,), jnp.int32),
            pltpu.VMEM((nblk + 16,), jnp.int32),
            pltpu.SemaphoreType.DMA,
            pltpu.SemaphoreType.DMA,
            pltpu.SemaphoreType.DMA,
            pltpu.SemaphoreType.DMA,
            pltpu.SemaphoreType.DMA,
            pltpu.SemaphoreType.DMA,
        ],
    )
    def k(rows_h, cols_h, vals_h, ck0_h, ck1_h, xw_h, out_h,
          acc, gbufA, gbufB, colA, colB, rowA, rowB, valA, valB,
          ck0_s, ck1_s, semiA, semiB, semcA, semcB, semgA, semgB):
        wid = lax.axis_index("s") * 2 + lax.axis_index("c")
        pltpu.async_copy(ck0_h, ck0_s.at[pl.ds(0, nblk)], semiA).wait()
        pltpu.async_copy(ck1_h, ck1_s.at[pl.ds(0, nblk)], semiA).wait()
        zz = jnp.zeros((16,), jnp.float32)
        bufs = ((gbufA, colA, rowA, valA, semiA, semcA, semgA),
                (gbufB, colB, rowB, valB, semiB, semcB, semgB))

        def issue(kk, bs):
            gbuf, colv, rowv, valv, semi, semc, semg = bs
            o = kk * CH
            pltpu.async_copy(cols_h.at[pl.ds(o, CH)], colv, semc).wait()
            pltpu.async_copy(rows_h.at[pl.ds(o, CH)],
                             rowv.at[pl.ds(0, CH)], semi)
            pltpu.async_copy(vals_h.at[pl.ds(o, CH)],
                             valv.at[pl.ds(0, CH)], semi)
            pltpu.make_async_copy(xw_h.at[colv.at[pl.ds(0, 128)]],
                                  gbuf.at[pl.ds(0, 128)], semg).start()
            pltpu.make_async_copy(xw_h.at[colv.at[pl.ds(128, 128)]],
                                  gbuf.at[pl.ds(128, 128)], semg).start()

        def compute(base, bs):
            gbuf, colv, rowv, valv, semi, semc, semg = bs
            pltpu.make_async_copy(rows_h.at[pl.ds(0, CH)],
                                  rowv.at[pl.ds(0, CH)], semi).wait()
            pltpu.make_async_copy(vals_h.at[pl.ds(0, CH)],
                                  valv.at[pl.ds(0, CH)], semi).wait()
            pltpu.make_async_copy(xw_h.at[colv.at[pl.ds(0, 128)]],
                                  gbuf.at[pl.ds(0, 128)], semg).wait()
            pltpu.make_async_copy(xw_h.at[colv.at[pl.ds(128, 128)]],
                                  gbuf.at[pl.ds(128, 128)], semg).wait()

            @pl.loop(0, CH, step=16)
            def _(q):
                rs16 = rowv[pl.ds(q, 16)]
                ok16 = (rs16 >= base) & (rs16 < base + rb)
                v16 = jnp.where(ok16, valv[pl.ds(q, 16)], 0.0)
                li16 = jnp.minimum(jnp.maximum(rs16 - base, 0), rb - 1)
                for i in range(16):
                    li = li16[i]
                    vv = jnp.full((16,), v16[i], jnp.float32)
                    acc[li, pl.ds(0, 16)] += gbuf[q + i, pl.ds(0, 16)] * vv
                    acc[li, pl.ds(16, 16)] += gbuf[q + i, pl.ds(16, 16)] * vv

        for t in range(nblk // NW):
            b = wid + t * NW
            base = b * rb

            @pl.loop(0, rb, step=8)
            def _(r):
                for i in range(8):
                    acc[r + i, pl.ds(0, 16)] = zz
                    acc[r + i, pl.ds(16, 16)] = zz

            c0 = ck0_s[pl.ds(b, 16)][0]
            c1 = ck1_s[pl.ds(b, 16)][0]

            @pl.when(c0 < c1)
            def _():
                issue(c0, bufs[0])

            @pl.loop(c0, c1, step=2)
            def _(kk0):
                @pl.when(kk0 + 1 < c1)
                def _():
                    issue(kk0 + 1, bufs[1])
                compute(base, bufs[0])

                @pl.when(kk0 + 1 < c1)
                def _():
                    @pl.when(kk0 + 2 < c1)
                    def _():
                        issue(kk0 + 2, bufs[0])
                    compute(base, bufs[1])

            pltpu.sync_copy(acc, out_h.at[pl.ds(base, rb)])

    return k


def _spmm(rows, cols, vals, xw, n_dst):
    nnz = rows.shape[0]
    if n_dst == N1:
        nblk, rb = (256, 632) if nnz > 600000 else (128, 1264)
    elif n_dst == N0:
        nblk, rb = 64, 160
    else:
        nblk, rb = 64, 784
    nnzp = ((nnz + CH - 1) // CH) * CH
    if nnzp != nnz:
        pad = nnzp - nnz
        rows = jnp.concatenate([rows, jnp.full((pad,), 1 << 28, jnp.int32)])
        cols = jnp.concatenate([cols, jnp.zeros((pad,), jnp.int32)])
        vals = jnp.concatenate([vals, jnp.zeros((pad,), jnp.float32)])
    off = jnp.searchsorted(rows, jnp.arange(nblk + 1, dtype=jnp.int32) * rb,
                           side="left").astype(jnp.int32)
    ck0 = off[:nblk] // CH
    ck1 = (off[1:] + CH - 1) // CH
    return _make_spmm(nnzp, rb, nblk)(rows, cols, vals, ck0, ck1, xw)


def _transform1(x, wcat, bcat, nops):
    n, kdim = x.shape

    def body(x_ref, w_ref, b_ref, *outs):
        y = jnp.dot(x_ref[...], w_ref[...], preferred_element_type=jnp.float32)
        y = y + b_ref[...]
        for i, o_ref in enumerate(outs):
            o_ref[...] = y[:, i * FP:(i + 1) * FP]

    grid = (pl.cdiv(n, ROWB),)
    return pl.pallas_call(
        body,
        grid=grid,
        in_specs=[
            pl.BlockSpec((ROWB, kdim), lambda i: (i, 0)),
            pl.BlockSpec((kdim, 4 * FP), lambda i: (0, 0)),
            pl.BlockSpec((1, 4 * FP), lambda i: (0, 0)),
        ],
        out_specs=[pl.BlockSpec((ROWB, FP), lambda i: (i, 0))] * nops,
        out_shape=[jax.ShapeDtypeStruct((n, FP), jnp.float32)] * nops,
    )(x, wcat, bcat)


def _combine_transform(parts, scale, wcat, bcat, nops):
    n = parts[0].shape[0]

    def body(*refs):
        ins = refs[:len(parts)]
        s = ins[0][...]
        for r in ins[1:len(parts)]:
            s = s + r[...]
        xc = scale * jnp.maximum(s, 0.0)
        if nops:
            w_ref = refs[len(parts)]
            b_ref = refs[len(parts) + 1]
            outs = refs[len(parts) + 2:]
            outs[0][...] = xc
            y = jnp.dot(xc, w_ref[...], preferred_element_type=jnp.float32)
            y = y + b_ref[...]
            for i in range(nops):
                outs[1 + i][...] = y[:, i * FP:(i + 1) * FP]
        else:
            refs[len(parts)][...] = xc

    grid = (n // ROWB,)
    in_specs = [pl.BlockSpec((ROWB, FP), lambda i: (i, 0))] * len(parts)
    if nops:
        in_specs += [
            pl.BlockSpec((FP, 4 * FP), lambda i: (0, 0)),
            pl.BlockSpec((1, 4 * FP), lambda i: (0, 0)),
        ]
        args = list(parts) + [wcat, bcat]
    else:
        args = list(parts)
    n_out = 1 + nops
    res = pl.pallas_call(
        body, grid=grid, in_specs=in_specs,
        out_specs=[pl.BlockSpec((ROWB, FP), lambda i: (i, 0))] * n_out,
        out_shape=[jax.ShapeDtypeStruct((n, FP), jnp.float32)] * n_out,
    )(*args)
    return (res[0], list(res[1:])) if nops else (res[0], [])


def _pool(xs, seg3d):
    n = xs[0].shape[0]

    def body(a_ref, b_ref, c_ref, s_ref, sums_ref, cnt_ref):
        seg = s_ref[0, 0, :]
        oh = (seg[:, None] == lax.broadcasted_iota(jnp.int32, (ROWB, B), 1))
        oh = oh.astype(jnp.float32)
        x = jnp.concatenate([a_ref[...], b_ref[...], c_ref[...]], axis=1)
        ps = lax.dot_general(oh, x, (((0,), (0,)), ((), ())),
                             preferred_element_type=jnp.float32)
        pc = lax.dot_general(oh, jnp.ones((ROWB, 8), jnp.float32),
                             (((0,), (0,)), ((), ())),
                             preferred_element_type=jnp.float32)

        @pl.when(pl.program_id(0) == 0)
        def _():
            sums_ref[...] = jnp.zeros_like(sums_ref)
            cnt_ref[...] = jnp.zeros_like(cnt_ref)

        sums_ref[...] += ps
        cnt_ref[...] += pc

    grid = (n // ROWB,)
    return pl.pallas_call(
        body,
        grid=grid,
        in_specs=[pl.BlockSpec((ROWB, FP), lambda i: (i, 0))] * 3
        + [pl.BlockSpec((1, 1, ROWB), lambda i: (i, 0, 0))],
        out_specs=[pl.BlockSpec((B, 3 * FP), lambda i: (0, 0)),
                   pl.BlockSpec((B, 8), lambda i: (0, 0))],
        out_shape=[jax.ShapeDtypeStruct((B, 3 * FP), jnp.float32),
                   jax.ShapeDtypeStruct((B, 8), jnp.float32)],
    )(*xs, seg3d)


def _head(sums, cnts, wc, bc, wo_parts, bo):
    def body(s0, s1, s2, c0, c1, c2, wc_ref, bc_ref, w0, w1, w2, bo_ref, o_ref):
        logits = bo_ref[...]
        for s_ref, c_ref, w_ref in ((s0, c0, w0), (s1, c1, w1), (s2, c2, w2)):
            cnt = jnp.maximum(c_ref[...][:, 0:1], 1.0)
            g = s_ref[...] / cnt
            p = jnp.dot(g, wc_ref[...], preferred_element_type=jnp.float32)
            p = p + bc_ref[...]
            logits = logits + jnp.dot(p, w_ref[...],
                                      preferred_element_type=jnp.float32)
        m = jnp.max(logits, axis=1, keepdims=True)
        e = jnp.exp(logits - m)
        o_ref[...] = e / jnp.sum(e, axis=1, keepdims=True)

    return pl.pallas_call(
        body,
        out_shape=jax.ShapeDtypeStruct((B, OUT), jnp.float32),
    )(sums[0], sums[1], sums[2], cnts[0], cnts[1], cnts[2], wc, bc,
      wo_parts[0], wo_parts[1], wo_parts[2], bo)


def _pack_w(ws, kdim):
    w = jnp.zeros((kdim, 4 * FP), jnp.float32)
    bvec = jnp.zeros((1, 4 * FP), jnp.float32)
    for i, (wi, bi) in enumerate(ws):
        kin = wi.shape[0]
        w = w.at[:kin, i * FP:i * FP + F].set(wi)
        bvec = bvec.at[0, i * FP:i * FP + F].set(bi)
    return w, bvec


def _pad_rows(x, npad):
    n = x.shape[0]
    if n == npad:
        return x
    return jnp.concatenate(
        [x, jnp.zeros((npad - n,) + x.shape[1:], x.dtype)], axis=0)


def kernel(X0, X1, X2, L0_rows, L0_cols, L0_vals, L1_rows, L1_cols, L1_vals,
           L2_rows, L2_cols, L2_vals, B2D3_rows, B2D3_cols, B2D3_vals,
           D2B1TD1inv_rows, D2B1TD1inv_cols, D2B1TD1inv_vals,
           D1invB1_rows, D1invB1_cols, D1invB1_vals,
           B2TD2inv_rows, B2TD2inv_cols, B2TD2inv_vals,
           batch0, batch1, batch2, params):
    p = params
    sp = {
        "L0": (L0_rows, L0_cols, L0_vals, N0),
        "L1": (L1_rows, L1_cols, L1_vals, N1),
        "L2": (L2_rows, L2_cols, L2_vals, N2),
        "B2D3": (B2D3_rows, B2D3_cols, B2D3_vals, N1),
        "D2B1TD1inv": (D2B1TD1inv_rows, D2B1TD1inv_cols, D2B1TD1inv_vals, N1),
        "D1invB1": (D1invB1_rows, D1invB1_cols, D1invB1_vals, N0),
        "B2TD2inv": (B2TD2inv_rows, B2TD2inv_cols, B2TD2inv_vals, N2),
    }

    def run_spmms(t0, t1, t2):
        n2n = _spmm(*sp["L0"][:3], t0[0], sp["L0"][3])
        n2e = _spmm(*sp["D2B1TD1inv"][:3], t0[1], sp["D2B1TD1inv"][3])
        e2n = _spmm(*sp["D1invB1"][:3], t1[0], sp["D1invB1"][3])
        e2e = _spmm(*sp["L1"][:3], t1[1], sp["L1"][3])
        e2t = _spmm(*sp["B2TD2inv"][:3], t1[2], sp["B2TD2inv"][3])
        t2t = _spmm(*sp["L2"][:3], t2[0], sp["L2"][3])
        t2e = _spmm(*sp["B2D3"][:3], t2[1], sp["B2D3"][3])
        return (n2n, e2n), (e2e, n2e, t2e), (t2t, e2t)

    def packed(layer, keys, kdim):
        return _pack_w([p[layer][k] for k in keys], kdim)

    w0, b0 = packed("l1", ("n2n", "n2e"), D)
    w1, b1 = packed("l1", ("e2n", "e2e", "e2t"), D)
    w2, b2 = packed("l1", ("t2t", "t2e"), D)
    t0 = _transform1(X0, w0, b0, 2)
    t1 = _transform1(X1, w1, b1, 3)
    t2 = _transform1(X2, w2, b2, 2)
    g0, g1, g2 = run_spmms(t0, t1, t2)

    xcs = []
    for layer in ("l2", "l3"):
        w0, b0 = packed(layer, ("n2n", "n2e"), FP)
        w1, b1 = packed(layer, ("e2n", "e2e", "e2t"), FP)
        w2, b2 = packed(layer, ("t2t", "t2e"), FP)
        xc0, t0 = _combine_transform(g0, 0.5, w0, b0, 2)
        xc1, t1 = _combine_transform(g1, 1.0 / 3.0, w1, b1, 3)
        xc2, t2 = _combine_transform(g2, 0.5, w2, b2, 2)
        xcs.append((xc0, xc1, xc2))
        g0, g1, g2 = run_spmms(t0, t1, t2)
    xc0_3, _ = _combine_transform(g0, 0.5, None, None, 0)
    xc1_3, _ = _combine_transform(g1, 1.0 / 3.0, None, None, 0)
    xc2_3, _ = _combine_transform(g2, 0.5, None, None, 0)
    xcs.append((xc0_3, xc1_3, xc2_3))

    def seg3d(batch, n, npad):
        s = jnp.concatenate([batch.astype(jnp.int32),
                             jnp.full((npad - n,), B, jnp.int32)])
        return s.reshape(npad // ROWB, 1, ROWB)

    sums, cnts = [], []
    for lvl, (batch, n) in enumerate(((batch0, N0), (batch1, N1), (batch2, N2))):
        npad = NPAD[n]
        xs = [xcs[0][lvl], xcs[1][lvl], xcs[2][lvl]]
        s, c = _pool(xs, seg3d(batch, n, npad))
        sums.append(s)
        cnts.append(c)

    wc_raw, bc_raw = p["combined1"]
    wc = jnp.zeros((3 * FP, OUT), jnp.float32)
    for i in range(3):
        wc = wc.at[i * FP:i * FP + F, :].set(wc_raw[i * F:(i + 1) * F, :])
    bc = bc_raw.reshape(1, OUT)
    wo_raw, bo_raw = p["output"]
    wo_parts = [wo_raw[i * OUT:(i + 1) * OUT, :] for i in range(3)]
    bo = bo_raw.reshape(1, OUT)
    return _head(sums, cnts, wc, bc, wo_parts, bo)


# speedup vs baseline: 1.0773x; 1.0000x over previous
import dataclasses
import functools

import jax
import jax.numpy as jnp
from jax import lax
from jax.experimental import pallas as pl
from jax.experimental.pallas import tpu as pltpu
from jax.experimental.pallas import tpu_sc as plsc

N0, N1, N2 = 10000, 160000, 50000
D, F, OUT, B = 256, 20, 64, 64
FP = 32
NW = 32
CH = 512
NPAD = {N0: 10240, N1: 161792, N2: 50176}
ROWB = 512


@functools.lru_cache(maxsize=None)
def _make_spmm(nnzp, rb, nblk):
    mesh = plsc.VectorSubcoreMesh(core_axis_name="c", subcore_axis_name="s")
    cp = pltpu.CompilerParams(use_tc_tiling_on_sc=False)

    @functools.partial(
        pl.kernel,
        out_type=jax.ShapeDtypeStruct((nblk * rb, FP), jnp.float32),
        mesh=mesh,
        compiler_params=cp,
        scratch_types=[
            pltpu.VMEM((rb, FP), jnp.float32),
            pltpu.VMEM((CH, FP), jnp.float32),
            pltpu.VMEM((CH, FP), jnp.float32),
            pltpu.VMEM((CH,), jnp.int32),
            pltpu.VMEM((CH,), jnp.int32),
            pltpu.VMEM((CH + 16,), jnp.int32),
            pltpu.VMEM((CH + 16,), jnp.int32),
            pltpu.VMEM((CH + 16,), jnp.float32),
            pltpu.VMEM((CH + 16,), jnp.float32),
            pltpu.VMEM((nblk + 16,), jnp.int32),
            pltpu.VMEM((nblk + 16,), jnp.int32),
            pltpu.SemaphoreType.DMA,
            pltpu.SemaphoreType.DMA,
            pltpu.SemaphoreType.DMA,
            pltpu.SemaphoreType.DMA,
            pltpu.SemaphoreType.DMA,
            pltpu.SemaphoreType.DMA,
        ],
    )
    def k(rows_h, cols_h, vals_h, ck0_h, ck1_h, xw_h, out_h,
          acc, gbufA, gbufB, colA, colB, rowA, rowB, valA, valB,
          ck0_s, ck1_s, semiA, semiB, semcA, semcB, semgA, semgB):
        wid = lax.axis_index("s") * 2 + lax.axis_index("c")
        pltpu.async_copy(ck0_h, ck0_s.at[pl.ds(0, nblk)], semiA).wait()
        pltpu.async_copy(ck1_h, ck1_s.at[pl.ds(0, nblk)], semiA).wait()
        zz = jnp.zeros((16,), jnp.float32)
        bufs = ((gbufA, colA, rowA, valA, semiA, semcA, semgA),
                (gbufB, colB, rowB, valB, semiB, semcB, semgB))

        def issue(kk, bs):
            gbuf, colv, rowv, valv, semi, semc, semg = bs
            o = kk * CH
            pltpu.async_copy(cols_h.at[pl.ds(o, CH)], colv, semc).wait()
            pltpu.async_copy(rows_h.at[pl.ds(o, CH)],
                             rowv.at[pl.ds(0, CH)], semi)
            pltpu.async_copy(vals_h.at[pl.ds(o, CH)],
                             valv.at[pl.ds(0, CH)], semi)
            for part in range(4):
                pltpu.make_async_copy(
                    xw_h.at[colv.at[pl.ds(part * 128, 128)]],
                    gbuf.at[pl.ds(part * 128, 128)], semg).start()

        def compute(base, bs):
            gbuf, colv, rowv, valv, semi, semc, semg = bs
            pltpu.make_async_copy(rows_h.at[pl.ds(0, CH)],
                                  rowv.at[pl.ds(0, CH)], semi).wait()
            pltpu.make_async_copy(vals_h.at[pl.ds(0, CH)],
                                  valv.at[pl.ds(0, CH)], semi).wait()
            for part in range(4):
                pltpu.make_async_copy(
                    xw_h.at[colv.at[pl.ds(part * 128, 128)]],
                    gbuf.at[pl.ds(part * 128, 128)], semg).wait()

            @pl.loop(0, CH, step=16)
            def _(q):
                rs16 = rowv[pl.ds(q, 16)]
                ok16 = (rs16 >= base) & (rs16 < base + rb)
                v16 = jnp.where(ok16, valv[pl.ds(q, 16)], 0.0)
                li16 = jnp.minimum(jnp.maximum(rs16 - base, 0), rb - 1)
                for i in range(16):
                    li = li16[i]
                    vv = jnp.full((16,), v16[i], jnp.float32)
                    acc[li, pl.ds(0, 16)] += gbuf[q + i, pl.ds(0, 16)] * vv
                    acc[li, pl.ds(16, 16)] += gbuf[q + i, pl.ds(16, 16)] * vv

        for t in range(nblk // NW):
            b = wid + t * NW
            base = b * rb

            @pl.loop(0, rb, step=8)
            def _(r):
                for i in range(8):
                    acc[r + i, pl.ds(0, 16)] = zz
                    acc[r + i, pl.ds(16, 16)] = zz

            c0 = ck0_s[pl.ds(b, 16)][0]
            c1 = ck1_s[pl.ds(b, 16)][0]

            @pl.when(c0 < c1)
            def _():
                issue(c0, bufs[0])

            @pl.loop(c0, c1, step=2)
            def _(kk0):
                @pl.when(kk0 + 1 < c1)
                def _():
                    issue(kk0 + 1, bufs[1])
                compute(base, bufs[0])

                @pl.when(kk0 + 1 < c1)
                def _():
                    @pl.when(kk0 + 2 < c1)
                    def _():
                        issue(kk0 + 2, bufs[0])
                    compute(base, bufs[1])

            pltpu.sync_copy(acc, out_h.at[pl.ds(base, rb)])

    return k


def _spmm(rows, cols, vals, xw, n_dst):
    nnz = rows.shape[0]
    if n_dst == N1:
        nblk, rb = (256, 632) if nnz > 600000 else (128, 1264)
    elif n_dst == N0:
        nblk, rb = 64, 160
    else:
        nblk, rb = 64, 784
    nnzp = ((nnz + CH - 1) // CH) * CH
    if nnzp != nnz:
        pad = nnzp - nnz
        rows = jnp.concatenate([rows, jnp.full((pad,), 1 << 28, jnp.int32)])
        cols = jnp.concatenate([cols, jnp.zeros((pad,), jnp.int32)])
        vals = jnp.concatenate([vals, jnp.zeros((pad,), jnp.float32)])
    off = jnp.searchsorted(rows, jnp.arange(nblk + 1, dtype=jnp.int32) * rb,
                           side="left").astype(jnp.int32)
    ck0 = off[:nblk] // CH
    ck1 = (off[1:] + CH - 1) // CH
    return _make_spmm(nnzp, rb, nblk)(rows, cols, vals, ck0, ck1, xw)


def _transform1(x, wcat, bcat, nops):
    n, kdim = x.shape

    def body(x_ref, w_ref, b_ref, *outs):
        y = jnp.dot(x_ref[...], w_ref[...], preferred_element_type=jnp.float32)
        y = y + b_ref[...]
        for i, o_ref in enumerate(outs):
            o_ref[...] = y[:, i * FP:(i + 1) * FP]

    grid = (pl.cdiv(n, ROWB),)
    return pl.pallas_call(
        body,
        grid=grid,
        in_specs=[
            pl.BlockSpec((ROWB, kdim), lambda i: (i, 0)),
            pl.BlockSpec((kdim, 4 * FP), lambda i: (0, 0)),
            pl.BlockSpec((1, 4 * FP), lambda i: (0, 0)),
        ],
        out_specs=[pl.BlockSpec((ROWB, FP), lambda i: (i, 0))] * nops,
        out_shape=[jax.ShapeDtypeStruct((n, FP), jnp.float32)] * nops,
    )(x, wcat, bcat)


def _combine_transform(parts, scale, wcat, bcat, nops):
    n = parts[0].shape[0]

    def body(*refs):
        ins = refs[:len(parts)]
        s = ins[0][...]
        for r in ins[1:len(parts)]:
            s = s + r[...]
        xc = scale * jnp.maximum(s, 0.0)
        if nops:
            w_ref = refs[len(parts)]
            b_ref = refs[len(parts) + 1]
            outs = refs[len(parts) + 2:]
            outs[0][...] = xc
            y = jnp.dot(xc, w_ref[...], preferred_element_type=jnp.float32)
            y = y + b_ref[...]
            for i in range(nops):
                outs[1 + i][...] = y[:, i * FP:(i + 1) * FP]
        else:
            refs[len(parts)][...] = xc

    grid = (n // ROWB,)
    in_specs = [pl.BlockSpec((ROWB, FP), lambda i: (i, 0))] * len(parts)
    if nops:
        in_specs += [
            pl.BlockSpec((FP, 4 * FP), lambda i: (0, 0)),
            pl.BlockSpec((1, 4 * FP), lambda i: (0, 0)),
        ]
        args = list(parts) + [wcat, bcat]
    else:
        args = list(parts)
    n_out = 1 + nops
    res = pl.pallas_call(
        body, grid=grid, in_specs=in_specs,
        out_specs=[pl.BlockSpec((ROWB, FP), lambda i: (i, 0))] * n_out,
        out_shape=[jax.ShapeDtypeStruct((n, FP), jnp.float32)] * n_out,
    )(*args)
    return (res[0], list(res[1:])) if nops else (res[0], [])


def _pool(xs, seg3d):
    n = xs[0].shape[0]

    def body(a_ref, b_ref, c_ref, s_ref, sums_ref, cnt_ref):
        seg = s_ref[0, 0, :]
        oh = (seg[:, None] == lax.broadcasted_iota(jnp.int32, (ROWB, B), 1))
        oh = oh.astype(jnp.float32)
        x = jnp.concatenate([a_ref[...], b_ref[...], c_ref[...]], axis=1)
        ps = lax.dot_general(oh, x, (((0,), (0,)), ((), ())),
                             preferred_element_type=jnp.float32)
        pc = lax.dot_general(oh, jnp.ones((ROWB, 8), jnp.float32),
                             (((0,), (0,)), ((), ())),
                             preferred_element_type=jnp.float32)

        @pl.when(pl.program_id(0) == 0)
        def _():
            sums_ref[...] = jnp.zeros_like(sums_ref)
            cnt_ref[...] = jnp.zeros_like(cnt_ref)

        sums_ref[...] += ps
        cnt_ref[...] += pc

    grid = (n // ROWB,)
    return pl.pallas_call(
        body,
        grid=grid,
        in_specs=[pl.BlockSpec((ROWB, FP), lambda i: (i, 0))] * 3
        + [pl.BlockSpec((1, 1, ROWB), lambda i: (i, 0, 0))],
        out_specs=[pl.BlockSpec((B, 3 * FP), lambda i: (0, 0)),
                   pl.BlockSpec((B, 8), lambda i: (0, 0))],
        out_shape=[jax.ShapeDtypeStruct((B, 3 * FP), jnp.float32),
                   jax.ShapeDtypeStruct((B, 8), jnp.float32)],
    )(*xs, seg3d)


def _head(sums, cnts, wc, bc, wo_parts, bo):
    def body(s0, s1, s2, c0, c1, c2, wc_ref, bc_ref, w0, w1, w2, bo_ref, o_ref):
        logits = bo_ref[...]
        for s_ref, c_ref, w_ref in ((s0, c0, w0), (s1, c1, w1), (s2, c2, w2)):
            cnt = jnp.maximum(c_ref[...][:, 0:1], 1.0)
            g = s_ref[...] / cnt
            p = jnp.dot(g, wc_ref[...], preferred_element_type=jnp.float32)
            p = p + bc_ref[...]
            logits = logits + jnp.dot(p, w_ref[...],
                                      preferred_element_type=jnp.float32)
        m = jnp.max(logits, axis=1, keepdims=True)
        e = jnp.exp(logits - m)
        o_ref[...] = e / jnp.sum(e, axis=1, keepdims=True)

    return pl.pallas_call(
        body,
        out_shape=jax.ShapeDtypeStruct((B, OUT), jnp.float32),
    )(sums[0], sums[1], sums[2], cnts[0], cnts[1], cnts[2], wc, bc,
      wo_parts[0], wo_parts[1], wo_parts[2], bo)


def _pack_w(ws, kdim):
    w = jnp.zeros((kdim, 4 * FP), jnp.float32)
    bvec = jnp.zeros((1, 4 * FP), jnp.float32)
    for i, (wi, bi) in enumerate(ws):
        kin = wi.shape[0]
        w = w.at[:kin, i * FP:i * FP + F].set(wi)
        bvec = bvec.at[0, i * FP:i * FP + F].set(bi)
    return w, bvec


def _pad_rows(x, npad):
    n = x.shape[0]
    if n == npad:
        return x
    return jnp.concatenate(
        [x, jnp.zeros((npad - n,) + x.shape[1:], x.dtype)], axis=0)


def kernel(X0, X1, X2, L0_rows, L0_cols, L0_vals, L1_rows, L1_cols, L1_vals,
           L2_rows, L2_cols, L2_vals, B2D3_rows, B2D3_cols, B2D3_vals,
           D2B1TD1inv_rows, D2B1TD1inv_cols, D2B1TD1inv_vals,
           D1invB1_rows, D1invB1_cols, D1invB1_vals,
           B2TD2inv_rows, B2TD2inv_cols, B2TD2inv_vals,
           batch0, batch1, batch2, params):
    p = params
    sp = {
        "L0": (L0_rows, L0_cols, L0_vals, N0),
        "L1": (L1_rows, L1_cols, L1_vals, N1),
        "L2": (L2_rows, L2_cols, L2_vals, N2),
        "B2D3": (B2D3_rows, B2D3_cols, B2D3_vals, N1),
        "D2B1TD1inv": (D2B1TD1inv_rows, D2B1TD1inv_cols, D2B1TD1inv_vals, N1),
        "D1invB1": (D1invB1_rows, D1invB1_cols, D1invB1_vals, N0),
        "B2TD2inv": (B2TD2inv_rows, B2TD2inv_cols, B2TD2inv_vals, N2),
    }

    def run_spmms(t0, t1, t2):
        n2n = _spmm(*sp["L0"][:3], t0[0], sp["L0"][3])
        n2e = _spmm(*sp["D2B1TD1inv"][:3], t0[1], sp["D2B1TD1inv"][3])
        e2n = _spmm(*sp["D1invB1"][:3], t1[0], sp["D1invB1"][3])
        e2e = _spmm(*sp["L1"][:3], t1[1], sp["L1"][3])
        e2t = _spmm(*sp["B2TD2inv"][:3], t1[2], sp["B2TD2inv"][3])
        t2t = _spmm(*sp["L2"][:3], t2[0], sp["L2"][3])
        t2e = _spmm(*sp["B2D3"][:3], t2[1], sp["B2D3"][3])
        return (n2n, e2n), (e2e, n2e, t2e), (t2t, e2t)

    def packed(layer, keys, kdim):
        return _pack_w([p[layer][k] for k in keys], kdim)

    w0, b0 = packed("l1", ("n2n", "n2e"), D)
    w1, b1 = packed("l1", ("e2n", "e2e", "e2t"), D)
    w2, b2 = packed("l1", ("t2t", "t2e"), D)
    t0 = _transform1(X0, w0, b0, 2)
    t1 = _transform1(X1, w1, b1, 3)
    t2 = _transform1(X2, w2, b2, 2)
    g0, g1, g2 = run_spmms(t0, t1, t2)

    xcs = []
    for layer in ("l2", "l3"):
        w0, b0 = packed(layer, ("n2n", "n2e"), FP)
        w1, b1 = packed(layer, ("e2n", "e2e", "e2t"), FP)
        w2, b2 = packed(layer, ("t2t", "t2e"), FP)
        xc0, t0 = _combine_transform(g0, 0.5, w0, b0, 2)
        xc1, t1 = _combine_transform(g1, 1.0 / 3.0, w1, b1, 3)
        xc2, t2 = _combine_transform(g2, 0.5, w2, b2, 2)
        xcs.append((xc0, xc1, xc2))
        g0, g1, g2 = run_spmms(t0, t1, t2)
    xc0_3, _ = _combine_transform(g0, 0.5, None, None, 0)
    xc1_3, _ = _combine_transform(g1, 1.0 / 3.0, None, None, 0)
    xc2_3, _ = _combine_transform(g2, 0.5, None, None, 0)
    xcs.append((xc0_3, xc1_3, xc2_3))

    def seg3d(batch, n, npad):
        s = jnp.concatenate([batch.astype(jnp.int32),
                             jnp.full((npad - n,), B, jnp.int32)])
        return s.reshape(npad // ROWB, 1, ROWB)

    sums, cnts = [], []
    for lvl, (batch, n) in enumerate(((batch0, N0), (batch1, N1), (batch2, N2))):
        npad = NPAD[n]
        xs = [xcs[0][lvl], xcs[1][lvl], xcs[2][lvl]]
        s, c = _pool(xs, seg3d(batch, n, npad))
        sums.append(s)
        cnts.append(c)

    wc_raw, bc_raw = p["combined1"]
    wc = jnp.zeros((3 * FP, OUT), jnp.float32)
    for i in range(3):
        wc = wc.at[i * FP:i * FP + F, :].set(wc_raw[i * F:(i + 1) * F, :])
    bc = bc_raw.reshape(1, OUT)
    wo_raw, bo_raw = p["output"]
    wo_parts = [wo_raw[i * OUT:(i + 1) * OUT, :] for i in range(3)]
    bo = bo_raw.reshape(1, OUT)
    return _head(sums, cnts, wc, bc, wo_parts, bo)
